# Initial kernel scaffold; baseline (speedup 1.0000x reference)
import jax
import jax.numpy as jnp
from jax.experimental import pallas as pl


def kernel(keypt, fixed_frame):
    raise NotImplementedError("write your pallas kernel here")


# speedup vs baseline: 6.6693x; 6.6693x over previous
import functools

import jax
import jax.numpy as jnp
from jax.experimental import pallas as pl

K_NN = 16
B = 4
N = 2048
ROWS = 256


def _loss_block(fx_nat_ref, fx_t_ref, kp_nat_ref, kp_t_ref, out_ref):
    i = pl.program_id(1)
    r0 = i * ROWS

    fxb = fx_t_ref[0]
    kpb = kp_t_ref[0]
    fx_rows = fx_nat_ref[0, pl.ds(r0, ROWS), :]
    kp_rows = kp_nat_ref[0, pl.ds(r0, ROWS), :]

    def sq_dist(rows, pts_t):
        a2 = jnp.sum(rows * rows, axis=1, keepdims=True)
        b2 = jnp.sum(pts_t * pts_t, axis=0, keepdims=True)
        ab = jax.lax.dot_general(
            rows, pts_t, (((1,), (0,)), ((), ())),
            preferred_element_type=jnp.float32,
            precision=jax.lax.Precision.HIGHEST)
        return jnp.maximum(a2 + b2 - 2.0 * ab, 0.0)

    d_fixed = sq_dist(fx_rows, fxb)
    d_kpt = sq_dist(kp_rows, kpb)

    iota = jax.lax.broadcasted_iota(jnp.int32, (ROWS, N), 1)
    loss = jnp.zeros((ROWS, 1), jnp.float32)
    d = d_fixed
    for _ in range(K_NN):
        m = jnp.min(d, axis=1, keepdims=True)
        eq = d == m
        j = jnp.min(jnp.where(eq, iota, N), axis=1, keepdims=True)
        onehot = iota == j
        dk = jnp.sum(jnp.where(onehot, d_kpt, 0.0), axis=1, keepdims=True)
        diff = m - dk
        loss = loss + diff * diff
        d = jnp.where(onehot, jnp.inf, d)

    out_ref[...] = jnp.sum(loss).reshape(1, 1, 1, 1)


def kernel(keypt, fixed_frame):
    fx_t = jnp.swapaxes(fixed_frame, 1, 2)
    kp_t = jnp.swapaxes(keypt, 1, 2)

    grid = (B, N // ROWS)
    partial = pl.pallas_call(
        _loss_block,
        grid=grid,
        in_specs=[
            pl.BlockSpec((1, N, 3), lambda b, i: (b, 0, 0)),
            pl.BlockSpec((1, 3, N), lambda b, i: (b, 0, 0)),
            pl.BlockSpec((1, N, 3), lambda b, i: (b, 0, 0)),
            pl.BlockSpec((1, 3, N), lambda b, i: (b, 0, 0)),
        ],
        out_specs=pl.BlockSpec((1, 1, 1, 1), lambda b, i: (b, i, 0, 0)),
        out_shape=jax.ShapeDtypeStruct((B, N // ROWS, 1, 1), jnp.float32),
    )(fixed_frame, fx_t, keypt, kp_t)

    return jnp.sum(partial) / (B * N)


# speedup vs baseline: 13.8941x; 2.0833x over previous
import functools

import jax
import jax.numpy as jnp
from jax.experimental import pallas as pl

K_NN = 16
B = 4
N = 2048
ROWS = 256


def _loss_block(fx_nat_ref, fx_t_ref, kp_nat_ref, kp_t_ref, out_ref):
    i = pl.program_id(1)
    r0 = i * ROWS

    fxb = fx_t_ref[0]
    kpb = kp_t_ref[0]
    fx_rows = fx_nat_ref[0, pl.ds(r0, ROWS), :]
    kp_rows = kp_nat_ref[0, pl.ds(r0, ROWS), :]

    def sq_dist(rows, pts_t):
        a2 = jnp.sum(rows * rows, axis=1, keepdims=True)
        b2 = jnp.sum(pts_t * pts_t, axis=0, keepdims=True)
        ab = jax.lax.dot_general(
            rows, pts_t, (((1,), (0,)), ((), ())),
            preferred_element_type=jnp.float32,
            precision=jax.lax.Precision.HIGHEST)
        return jnp.maximum(a2 + b2 - 2.0 * ab, 0.0)

    d_fixed = sq_dist(fx_rows, fxb)
    d_kpt = sq_dist(kp_rows, kpb)

    diff = d_fixed - d_kpt
    s = diff * diff
    d = d_fixed
    m = jnp.min(d, axis=1, keepdims=True)
    for _ in range(K_NN - 1):
        d = jnp.where(d <= m, jnp.inf, d)
        m = jnp.min(d, axis=1, keepdims=True)

    lt = d_fixed < m
    eqm = d_fixed == m
    s_lt = jnp.sum(jnp.where(lt, s, 0.0), axis=1, keepdims=True)
    s_eq = jnp.sum(jnp.where(eqm, s, 0.0), axis=1, keepdims=True)
    cnt_lt = jnp.sum(jnp.where(lt, 1.0, 0.0), axis=1, keepdims=True)
    cnt_eq = jnp.sum(jnp.where(eqm, 1.0, 0.0), axis=1, keepdims=True)
    w = jnp.clip((K_NN - cnt_lt) / jnp.maximum(cnt_eq, 1.0), 0.0, 1.0)
    loss = jnp.sum(s_lt + w * s_eq)
    out_ref[...] = loss.reshape(1, 1, 1, 1)


def kernel(keypt, fixed_frame):
    fx_t = jnp.swapaxes(fixed_frame, 1, 2)
    kp_t = jnp.swapaxes(keypt, 1, 2)

    grid = (B, N // ROWS)
    partial = pl.pallas_call(
        _loss_block,
        grid=grid,
        in_specs=[
            pl.BlockSpec((1, N, 3), lambda b, i: (b, 0, 0)),
            pl.BlockSpec((1, 3, N), lambda b, i: (b, 0, 0)),
            pl.BlockSpec((1, N, 3), lambda b, i: (b, 0, 0)),
            pl.BlockSpec((1, 3, N), lambda b, i: (b, 0, 0)),
        ],
        out_specs=pl.BlockSpec((1, 1, 1, 1), lambda b, i: (b, i, 0, 0)),
        out_shape=jax.ShapeDtypeStruct((B, N // ROWS, 1, 1), jnp.float32),
    )(fixed_frame, fx_t, keypt, kp_t)

    return jnp.sum(partial) / (B * N)


# speedup vs baseline: 14.0018x; 1.0078x over previous
import functools

import jax
import jax.numpy as jnp
from jax.experimental import pallas as pl

K_NN = 16
B = 4
N = 2048
ROWS = 256


def _loss_block(fx_nat_ref, fx_t_ref, kp_nat_ref, kp_t_ref, out_ref):
    i = pl.program_id(1)
    r0 = i * ROWS

    fxb = fx_t_ref[0]
    kpb = kp_t_ref[0]
    fx_rows = fx_nat_ref[0, pl.ds(r0, ROWS), :]
    kp_rows = kp_nat_ref[0, pl.ds(r0, ROWS), :]

    def sq_dist(rows, pts_t):
        a2 = jnp.sum(rows * rows, axis=1, keepdims=True)
        b2 = jnp.sum(pts_t * pts_t, axis=0, keepdims=True)
        ab = jax.lax.dot_general(
            rows, pts_t, (((1,), (0,)), ((), ())),
            preferred_element_type=jnp.float32,
            precision=jax.lax.Precision.HIGHEST)
        return jnp.maximum(a2 + b2 - 2.0 * ab, 0.0)

    d_fixed = sq_dist(fx_rows, fxb)
    d_kpt = sq_dist(kp_rows, kpb)

    m = jnp.min(d_fixed, axis=1, keepdims=True)
    for _ in range(K_NN - 1):
        m = jnp.min(jnp.where(d_fixed > m, d_fixed, jnp.inf),
                    axis=1, keepdims=True)

    diff = d_fixed - d_kpt
    s = diff * diff
    le = d_fixed <= m
    eqm = d_fixed == m
    s_le = jnp.sum(jnp.where(le, s, 0.0), axis=1, keepdims=True)
    s_eq = jnp.sum(jnp.where(eqm, s, 0.0), axis=1, keepdims=True)
    cnt_le = jnp.sum(jnp.where(le, 1.0, 0.0), axis=1, keepdims=True)
    cnt_eq = jnp.sum(jnp.where(eqm, 1.0, 0.0), axis=1, keepdims=True)
    cnt_lt = cnt_le - cnt_eq
    w = jnp.clip((K_NN - cnt_lt) / jnp.maximum(cnt_eq, 1.0), 0.0, 1.0)
    loss = jnp.sum(s_le - (1.0 - w) * s_eq)
    out_ref[...] = loss.reshape(1, 1, 1, 1)


def kernel(keypt, fixed_frame):
    fx_t = jnp.swapaxes(fixed_frame, 1, 2)
    kp_t = jnp.swapaxes(keypt, 1, 2)

    grid = (B, N // ROWS)
    partial = pl.pallas_call(
        _loss_block,
        grid=grid,
        in_specs=[
            pl.BlockSpec((1, N, 3), lambda b, i: (b, 0, 0)),
            pl.BlockSpec((1, 3, N), lambda b, i: (b, 0, 0)),
            pl.BlockSpec((1, N, 3), lambda b, i: (b, 0, 0)),
            pl.BlockSpec((1, 3, N), lambda b, i: (b, 0, 0)),
        ],
        out_specs=pl.BlockSpec((1, 1, 1, 1), lambda b, i: (b, i, 0, 0)),
        out_shape=jax.ShapeDtypeStruct((B, N // ROWS, 1, 1), jnp.float32),
    )(fixed_frame, fx_t, keypt, kp_t)

    return jnp.sum(partial) / (B * N)


# speedup vs baseline: 14.1239x; 1.0087x over previous
import functools

import jax
import jax.numpy as jnp
from jax.experimental import pallas as pl

K_NN = 16
B = 4
N = 2048
ROWS = 512


def _loss_block(fx_nat_ref, fx_t_ref, kp_nat_ref, kp_t_ref, out_ref):
    i = pl.program_id(1)
    r0 = i * ROWS

    fxb = fx_t_ref[0]
    kpb = kp_t_ref[0]
    fx_rows = fx_nat_ref[0, pl.ds(r0, ROWS), :]
    kp_rows = kp_nat_ref[0, pl.ds(r0, ROWS), :]

    def sq_dist(rows, pts_t):
        a2 = jnp.sum(rows * rows, axis=1, keepdims=True)
        b2 = jnp.sum(pts_t * pts_t, axis=0, keepdims=True)
        ab = jax.lax.dot_general(
            rows, pts_t, (((1,), (0,)), ((), ())),
            preferred_element_type=jnp.float32,
            precision=jax.lax.Precision.HIGHEST)
        return jnp.maximum(a2 + b2 - 2.0 * ab, 0.0)

    d_fixed = sq_dist(fx_rows, fxb)
    d_kpt = sq_dist(kp_rows, kpb)

    m = jnp.min(d_fixed, axis=1, keepdims=True)
    for _ in range(K_NN - 1):
        m = jnp.min(jnp.where(d_fixed > m, d_fixed, jnp.inf),
                    axis=1, keepdims=True)

    diff = d_fixed - d_kpt
    s = diff * diff
    le = d_fixed <= m
    eqm = d_fixed == m
    s_le = jnp.sum(jnp.where(le, s, 0.0), axis=1, keepdims=True)
    s_eq = jnp.sum(jnp.where(eqm, s, 0.0), axis=1, keepdims=True)
    cnt_le = jnp.sum(jnp.where(le, 1.0, 0.0), axis=1, keepdims=True)
    cnt_eq = jnp.sum(jnp.where(eqm, 1.0, 0.0), axis=1, keepdims=True)
    cnt_lt = cnt_le - cnt_eq
    w = jnp.clip((K_NN - cnt_lt) / jnp.maximum(cnt_eq, 1.0), 0.0, 1.0)
    loss = jnp.sum(s_le - (1.0 - w) * s_eq)
    out_ref[...] = loss.reshape(1, 1, 1, 1)


def kernel(keypt, fixed_frame):
    fx_t = jnp.swapaxes(fixed_frame, 1, 2)
    kp_t = jnp.swapaxes(keypt, 1, 2)

    grid = (B, N // ROWS)
    partial = pl.pallas_call(
        _loss_block,
        grid=grid,
        in_specs=[
            pl.BlockSpec((1, N, 3), lambda b, i: (b, 0, 0)),
            pl.BlockSpec((1, 3, N), lambda b, i: (b, 0, 0)),
            pl.BlockSpec((1, N, 3), lambda b, i: (b, 0, 0)),
            pl.BlockSpec((1, 3, N), lambda b, i: (b, 0, 0)),
        ],
        out_specs=pl.BlockSpec((1, 1, 1, 1), lambda b, i: (b, i, 0, 0)),
        out_shape=jax.ShapeDtypeStruct((B, N // ROWS, 1, 1), jnp.float32),
    )(fixed_frame, fx_t, keypt, kp_t)

    return jnp.sum(partial) / (B * N)
